# Initial kernel scaffold; baseline (speedup 1.0000x reference)
import functools

import jax
import jax.numpy as jnp
from jax import lax
from jax.experimental import pallas as pl
from jax.experimental.pallas import tpu as pltpu
from jax.experimental.pallas import tpu_sc as plsc

BATCH = 4096
HIST = 50
EMBED_DIM = 64

NUM_CORES = 2
NUM_SUBCORES = 16
NUM_WORKERS = NUM_CORES * NUM_SUBCORES

TOTAL = BATCH * HIST
PER_WORKER = TOTAL // NUM_WORKERS
CHUNK = 128
NCHUNK = PER_WORKER // CHUNK


@jax.jit
def _sc_embedding_lookup(ids_grouped, table):
    mesh = plsc.VectorSubcoreMesh(
        core_axis_name="c", subcore_axis_name="s",
        num_cores=NUM_CORES, num_subcores=NUM_SUBCORES)

    @functools.partial(
        pl.kernel,
        out_type=jax.ShapeDtypeStruct((NUM_WORKERS, NCHUNK, CHUNK, EMBED_DIM),
                                      jnp.float32),
        mesh=mesh,
        scratch_types=[
            pltpu.VMEM((NCHUNK, CHUNK), jnp.int32),
            pltpu.VMEM((CHUNK, EMBED_DIM), jnp.float32),
            pltpu.SemaphoreType.DMA,
        ],
    )
    def k(ids_hbm, table_hbm, out_hbm, idx_v, rows_v, gsem):
        wid = lax.axis_index("s") * NUM_CORES + lax.axis_index("c")
        pltpu.sync_copy(ids_hbm.at[wid], idx_v)

        def body(j, _):
            pltpu.async_copy(table_hbm.at[idx_v.at[j]], rows_v, gsem).wait()
            pltpu.sync_copy(rows_v, out_hbm.at[wid, j])
            return 0

        lax.fori_loop(0, NCHUNK, body, 0)

    return k(ids_grouped, table)


def kernel(satellite_ids, embedding_table):
    ids_grouped = satellite_ids.reshape(NUM_WORKERS, NCHUNK, CHUNK)
    out = _sc_embedding_lookup(ids_grouped, embedding_table)
    return out.reshape(BATCH, HIST, EMBED_DIM)


# speedup vs baseline: 4.0830x; 4.0830x over previous
import functools

import jax
import jax.numpy as jnp
from jax import lax
from jax.experimental import pallas as pl
from jax.experimental.pallas import tpu as pltpu
from jax.experimental.pallas import tpu_sc as plsc

BATCH = 4096
HIST = 50
EMBED_DIM = 64

NUM_CORES = 2
NUM_SUBCORES = 16
NUM_WORKERS = NUM_CORES * NUM_SUBCORES

TOTAL = BATCH * HIST
PER_WORKER = TOTAL // NUM_WORKERS
CHUNK = 128
NCHUNK = PER_WORKER // CHUNK


@jax.jit
def _sc_embedding_lookup(ids_grouped, table):
    mesh = plsc.VectorSubcoreMesh(
        core_axis_name="c", subcore_axis_name="s",
        num_cores=NUM_CORES, num_subcores=NUM_SUBCORES)

    @functools.partial(
        pl.kernel,
        out_type=jax.ShapeDtypeStruct((NUM_WORKERS, NCHUNK, CHUNK, EMBED_DIM),
                                      jnp.float32),
        mesh=mesh,
        scratch_types=[
            pltpu.VMEM((NCHUNK, CHUNK), jnp.int32),
            pltpu.VMEM((CHUNK, EMBED_DIM), jnp.float32),
            pltpu.SemaphoreType.DMA,
        ],
        compiler_params=pltpu.CompilerParams(use_tc_tiling_on_sc=False),
    )
    def k(ids_hbm, table_hbm, out_hbm, idx_v, rows_v, gsem):
        wid = lax.axis_index("s") * NUM_CORES + lax.axis_index("c")
        pltpu.sync_copy(ids_hbm.at[wid], idx_v)

        def body(j, _):
            pltpu.async_copy(table_hbm.at[idx_v.at[j]], rows_v, gsem).wait()
            pltpu.sync_copy(rows_v, out_hbm.at[wid, j])
            return 0

        lax.fori_loop(0, NCHUNK, body, 0)

    return k(ids_grouped, table)


def kernel(satellite_ids, embedding_table):
    ids_grouped = satellite_ids.reshape(NUM_WORKERS, NCHUNK, CHUNK)
    out = _sc_embedding_lookup(ids_grouped, embedding_table)
    return out.reshape(BATCH, HIST, EMBED_DIM)


# speedup vs baseline: 4.6636x; 1.1422x over previous
import functools

import jax
import jax.numpy as jnp
from jax import lax
from jax.experimental import pallas as pl
from jax.experimental.pallas import tpu as pltpu
from jax.experimental.pallas import tpu_sc as plsc

BATCH = 4096
HIST = 50
EMBED_DIM = 64

NUM_CORES = 2
NUM_SUBCORES = 16
NUM_WORKERS = NUM_CORES * NUM_SUBCORES

TOTAL = BATCH * HIST
PER_WORKER = TOTAL // NUM_WORKERS
CHUNK = 128
NCHUNK = PER_WORKER // CHUNK
NBUF = 5


@jax.jit
def _sc_embedding_lookup(ids_grouped, table):
    mesh = plsc.VectorSubcoreMesh(
        core_axis_name="c", subcore_axis_name="s",
        num_cores=NUM_CORES, num_subcores=NUM_SUBCORES)

    @functools.partial(
        pl.kernel,
        out_type=jax.ShapeDtypeStruct((NUM_WORKERS, NCHUNK, CHUNK, EMBED_DIM),
                                      jnp.float32),
        mesh=mesh,
        scratch_types=[
            pltpu.VMEM((NCHUNK, CHUNK), jnp.int32),
            pltpu.VMEM((NBUF, CHUNK, EMBED_DIM), jnp.float32),
            pltpu.SemaphoreType.DMA((NBUF,)),
            pltpu.SemaphoreType.DMA((NBUF,)),
        ],
        compiler_params=pltpu.CompilerParams(use_tc_tiling_on_sc=False),
    )
    def k(ids_hbm, table_hbm, out_hbm, idx_v, rows_v, gsems, wsems):
        wid = lax.axis_index("s") * NUM_CORES + lax.axis_index("c")
        pltpu.sync_copy(ids_hbm.at[wid], idx_v)

        for b in range(NBUF):
            pltpu.async_copy(table_hbm.at[idx_v.at[b]], rows_v.at[b],
                             gsems.at[b])

        def outer(g, _):
            for b in range(NBUF):
                j = g * NBUF + b
                pltpu.make_async_copy(table_hbm.at[idx_v.at[b]],
                                      rows_v.at[b], gsems.at[b]).wait()
                pltpu.async_copy(rows_v.at[b], out_hbm.at[wid, j],
                                 wsems.at[b])
            for b in range(NBUF):
                j = g * NBUF + b

                @pl.when(j + NBUF < NCHUNK)
                def _():
                    pltpu.make_async_copy(rows_v.at[b], out_hbm.at[wid, j],
                                          wsems.at[b]).wait()
                    pltpu.async_copy(table_hbm.at[idx_v.at[j + NBUF]],
                                     rows_v.at[b], gsems.at[b])

            return 0

        lax.fori_loop(0, NCHUNK // NBUF, outer, 0)

        for b in range(NBUF):
            pltpu.make_async_copy(rows_v.at[b],
                                  out_hbm.at[wid, NCHUNK - NBUF + b],
                                  wsems.at[b]).wait()

    return k(ids_grouped, table)


def kernel(satellite_ids, embedding_table):
    ids_grouped = satellite_ids.reshape(NUM_WORKERS, NCHUNK, CHUNK)
    out = _sc_embedding_lookup(ids_grouped, embedding_table)
    return out.reshape(BATCH, HIST, EMBED_DIM)
